# baseline (speedup 1.0000x reference)
import functools

import numpy as np
import jax
import jax.numpy as jnp
from jax.experimental import pallas as pl

T = 8192
HIDDEN = 2048
NHEADS = 16
HEAD_DIM = 128
ROPE_DIM = 64
QLORA = 1536
ROPE_THETA = 10000.0

BM = 256


def _hadamard128():
    h = np.array([[1.0]], dtype=np.float64)
    while h.shape[0] < HEAD_DIM:
        h = np.block([[h, h], [h, -h]])
    return h.astype(np.float32)


_H128 = _hadamard128()

_PERM_HALF = np.concatenate(
    [np.arange(0, ROPE_DIM, 2), np.arange(1, ROPE_DIM, 2), np.arange(ROPE_DIM, HEAD_DIM)]
)
_INV_FREQ = (
    1.0 / (ROPE_THETA ** (np.arange(0, ROPE_DIM, 2).astype(np.float32) / ROPE_DIM))
).reshape(1, ROPE_DIM // 2)


def _indexer_kernel(ql_ref, hid_ref, cos_ref, sin_ref, wq_ref, wk_ref,
                    gam_ref, bet_ref, hmat_ref, q_out_ref, k_out_ref):
    half = ROPE_DIM // 2
    bm = ql_ref.shape[0]
    cos = cos_ref[...]
    sin = sin_ref[...]
    hmat = hmat_ref[...]
    ones64 = jnp.ones((bm, HEAD_DIM - ROPE_DIM), jnp.float32)
    zeros32 = jnp.zeros((bm, half), jnp.float32)
    zeros64 = jnp.zeros((bm, HEAD_DIM - ROPE_DIM), jnp.float32)
    head_c = jnp.concatenate([cos, cos, ones64], axis=1)
    head_sl = jnp.concatenate([-sin, zeros32, zeros64], axis=1)
    head_sr = jnp.concatenate([zeros32, sin, zeros64], axis=1)

    def rope(x):
        reps = x.shape[1] // HEAD_DIM
        c = jnp.concatenate([head_c] * reps, axis=1) if reps > 1 else head_c
        sl = jnp.concatenate([head_sl] * reps, axis=1) if reps > 1 else head_sl
        sr = jnp.concatenate([head_sr] * reps, axis=1) if reps > 1 else head_sr
        xl = jnp.concatenate([x[:, half:], x[:, :half]], axis=1)
        xr = jnp.concatenate([x[:, -half:], x[:, :-half]], axis=1)
        return x * c + xl * sl + xr * sr

    k = jnp.dot(hid_ref[...].astype(jnp.bfloat16), wk_ref[...],
                preferred_element_type=jnp.float32)
    mu = jnp.mean(k, axis=1, keepdims=True)
    var = jnp.mean((k - mu) ** 2, axis=1, keepdims=True)
    k = (k - mu) * jax.lax.rsqrt(var + 1e-5) * gam_ref[...] + bet_ref[...]
    k_out_ref[...] = jnp.dot(rope(k).astype(jnp.bfloat16), hmat,
                             preferred_element_type=jnp.float32) * (HEAD_DIM ** -0.5)

    q = jnp.dot(ql_ref[...].astype(jnp.bfloat16), wq_ref[...],
                preferred_element_type=jnp.float32)
    qr = rope(q).astype(jnp.bfloat16)
    heads = []
    for h in range(NHEADS):
        heads.append(
            jnp.dot(qr[:, h * HEAD_DIM:(h + 1) * HEAD_DIM], hmat,
                    preferred_element_type=jnp.float32) * (HEAD_DIM ** -0.5))
    q_out_ref[...] = jnp.concatenate(heads, axis=1)


@jax.jit
def kernel(q_lora, hidden_states, positions, Wq_b, Wk, k_gamma, k_beta):
    nt = q_lora.shape[0]
    qperm = (np.arange(NHEADS)[:, None] * HEAD_DIM + _PERM_HALF[None, :]).reshape(-1)
    wq = Wq_b[:, qperm].astype(jnp.bfloat16)
    wk = Wk[:, _PERM_HALF].astype(jnp.bfloat16)
    gam = k_gamma[_PERM_HALF].reshape(1, HEAD_DIM)
    bet = k_beta[_PERM_HALF].reshape(1, HEAD_DIM)
    freqs = positions.astype(jnp.float32)[:, None] * jnp.asarray(_INV_FREQ)
    cos_t = jnp.cos(freqs)
    sin_t = jnp.sin(freqs)

    grid = (nt // BM,)
    q2d, key = pl.pallas_call(
        _indexer_kernel,
        grid=grid,
        in_specs=[
            pl.BlockSpec((BM, QLORA), lambda i: (i, 0)),
            pl.BlockSpec((BM, HIDDEN), lambda i: (i, 0)),
            pl.BlockSpec((BM, ROPE_DIM // 2), lambda i: (i, 0)),
            pl.BlockSpec((BM, ROPE_DIM // 2), lambda i: (i, 0)),
            pl.BlockSpec((QLORA, NHEADS * HEAD_DIM), lambda i: (0, 0)),
            pl.BlockSpec((HIDDEN, HEAD_DIM), lambda i: (0, 0)),
            pl.BlockSpec((1, HEAD_DIM), lambda i: (0, 0)),
            pl.BlockSpec((1, HEAD_DIM), lambda i: (0, 0)),
            pl.BlockSpec((HEAD_DIM, HEAD_DIM), lambda i: (0, 0)),
        ],
        out_specs=[
            pl.BlockSpec((BM, NHEADS * HEAD_DIM), lambda i: (i, 0)),
            pl.BlockSpec((BM, HEAD_DIM), lambda i: (i, 0)),
        ],
        out_shape=[
            jax.ShapeDtypeStruct((nt, NHEADS * HEAD_DIM), jnp.float32),
            jax.ShapeDtypeStruct((nt, HEAD_DIM), jnp.float32),
        ],
    )(q_lora, hidden_states, cos_t, sin_t, wq, wk, gam, bet,
      jnp.asarray(_H128, dtype=jnp.bfloat16))
    return q2d.reshape(nt, NHEADS, HEAD_DIM), key


# speedup vs baseline: 1.3853x; 1.3853x over previous
import numpy as np
import jax
import jax.numpy as jnp
from jax.experimental import pallas as pl
from jax.experimental.pallas import tpu as pltpu

T = 8192
HIDDEN = 2048
NHEADS = 16
HEAD_DIM = 128
ROPE_DIM = 64
QLORA = 1536
ROPE_THETA = 10000.0

BM = 256


def _hadamard_permuted():
    h = np.array([[1.0]], dtype=np.float64)
    while h.shape[0] < HEAD_DIM:
        h = np.block([[h, h], [h, -h]])
    inv = np.arange(HEAD_DIM)
    i = np.arange(ROPE_DIM)
    inv[:ROPE_DIM] = np.where(i % 2 == 0, i // 2, ROPE_DIM // 2 + i // 2)
    return h[inv].astype(np.float32)


_H128P = _hadamard_permuted()
_INV_FREQ = (
    1.0 / (ROPE_THETA ** (np.arange(0, ROPE_DIM, 2).astype(np.float32) / ROPE_DIM))
).reshape(1, ROPE_DIM // 2)
_HSCALE = HEAD_DIM ** -0.5


def _indexer_kernel(ql_ref, hid_ref, c_ref, sl_ref, sr_ref, wq_ref, wk_ref,
                    gam_ref, bet_ref, hmat_ref, q_out_ref, k_out_ref,
                    wq_bf_ref, wk_bf_ref):
    @pl.when(pl.program_id(0) == 0)
    def _cache_weights():
        wq_bf_ref[...] = wq_ref[...].astype(jnp.bfloat16)
        wk_bf_ref[...] = wk_ref[...].astype(jnp.bfloat16)

    c1 = c_ref[...]
    sl1 = sl_ref[...]
    sr1 = sr_ref[...]
    hmat = hmat_ref[...]

    def rope_then_h(x):
        xl = jnp.concatenate([x[:, 1:], x[:, :1]], axis=1)
        xr = jnp.concatenate([x[:, -1:], x[:, :-1]], axis=1)
        rot = x * c1 + xl * sl1 + xr * sr1
        return jnp.dot(rot.astype(jnp.bfloat16), hmat,
                       preferred_element_type=jnp.float32)

    k = jnp.dot(hid_ref[...].astype(jnp.bfloat16), wk_bf_ref[...],
                preferred_element_type=jnp.float32)
    mu = jnp.mean(k, axis=1, keepdims=True)
    var = jnp.mean((k - mu) ** 2, axis=1, keepdims=True)
    k = (k - mu) * jax.lax.rsqrt(var + 1e-5) * gam_ref[...] + bet_ref[...]
    k_out_ref[...] = rope_then_h(k)

    q = jnp.dot(ql_ref[...].astype(jnp.bfloat16), wq_bf_ref[...],
                preferred_element_type=jnp.float32)
    for h in range(NHEADS):
        q_out_ref[:, h, :] = rope_then_h(q[:, h * HEAD_DIM:(h + 1) * HEAD_DIM])


@jax.jit
def kernel(q_lora, hidden_states, positions, Wq_b, Wk, k_gamma, k_beta):
    nt = q_lora.shape[0]
    freqs = positions.astype(jnp.float32)[:, None] * jnp.asarray(_INV_FREQ)
    cos = jnp.cos(freqs)
    sin = jnp.sin(freqs)
    z32 = jnp.zeros_like(sin)
    pad = jnp.full((nt, HEAD_DIM - ROPE_DIM), _HSCALE, jnp.float32)
    zpad = jnp.zeros((nt, HEAD_DIM - ROPE_DIM), jnp.float32)
    c_t = jnp.concatenate(
        [jnp.stack([cos, cos], axis=-1).reshape(nt, ROPE_DIM) * _HSCALE, pad],
        axis=1)
    sl_t = jnp.concatenate(
        [jnp.stack([-sin, z32], axis=-1).reshape(nt, ROPE_DIM) * _HSCALE, zpad],
        axis=1)
    sr_t = jnp.concatenate(
        [jnp.stack([z32, sin], axis=-1).reshape(nt, ROPE_DIM) * _HSCALE, zpad],
        axis=1)
    gam = k_gamma.reshape(1, HEAD_DIM)
    bet = k_beta.reshape(1, HEAD_DIM)

    grid = (nt // BM,)
    q2d, key = pl.pallas_call(
        _indexer_kernel,
        grid=grid,
        in_specs=[
            pl.BlockSpec((BM, QLORA), lambda i: (i, 0)),
            pl.BlockSpec((BM, HIDDEN), lambda i: (i, 0)),
            pl.BlockSpec((BM, HEAD_DIM), lambda i: (i, 0)),
            pl.BlockSpec((BM, HEAD_DIM), lambda i: (i, 0)),
            pl.BlockSpec((BM, HEAD_DIM), lambda i: (i, 0)),
            pl.BlockSpec((QLORA, NHEADS * HEAD_DIM), lambda i: (0, 0)),
            pl.BlockSpec((HIDDEN, HEAD_DIM), lambda i: (0, 0)),
            pl.BlockSpec((1, HEAD_DIM), lambda i: (0, 0)),
            pl.BlockSpec((1, HEAD_DIM), lambda i: (0, 0)),
            pl.BlockSpec((HEAD_DIM, HEAD_DIM), lambda i: (0, 0)),
        ],
        out_specs=[
            pl.BlockSpec((BM, NHEADS, HEAD_DIM), lambda i: (i, 0, 0)),
            pl.BlockSpec((BM, HEAD_DIM), lambda i: (i, 0)),
        ],
        out_shape=[
            jax.ShapeDtypeStruct((nt, NHEADS, HEAD_DIM), jnp.float32),
            jax.ShapeDtypeStruct((nt, HEAD_DIM), jnp.float32),
        ],
        scratch_shapes=[
            pltpu.VMEM((QLORA, NHEADS * HEAD_DIM), jnp.bfloat16),
            pltpu.VMEM((HIDDEN, HEAD_DIM), jnp.bfloat16),
        ],
    )(q_lora, hidden_states, c_t, sl_t, sr_t, Wq_b, Wk, gam, bet,
      jnp.asarray(_H128P, dtype=jnp.bfloat16))
    return q2d, key
